# baseline (speedup 1.0000x reference)
import jax
import jax.numpy as jnp
from jax import lax
from jax.experimental import pallas as pl
from jax.experimental.pallas import tpu as pltpu
from jax.experimental.pallas import tpu_sc as plsc

N_NODES = 10000
N_EDGES = 320000
D = 128

NC = 2
NS = 16
L = 16

CHUNK = 128
CH_PER_TILE = 80
E_PAD = NC * NS * CH_PER_TILE * CHUNK

N_PAD = 10240
ROW_CHUNK = 128
ROW_CHUNKS_PER_SUB = N_PAD // NS // ROW_CHUNK


def _make_edge_agg(use_cids: bool):
    mesh = plsc.VectorSubcoreMesh(
        core_axis_name="c", subcore_axis_name="s", num_cores=NC, num_subcores=NS
    )

    scratch = [
        pltpu.VMEM_SHARED((N_PAD, D), jnp.float32),
        pltpu.VMEM((CHUNK,), jnp.int32),
        pltpu.VMEM((CHUNK,), jnp.int32),
        pltpu.VMEM((CHUNK,), jnp.int32),
        pltpu.VMEM((CHUNK,), jnp.int32),
        pltpu.VMEM((CHUNK, D), jnp.float32),
        pltpu.VMEM((CHUNK, D), jnp.float32),
        pltpu.SemaphoreType.DMA,
        pltpu.SemaphoreType.DMA,
    ]
    if use_cids:
        scratch.insert(1, pltpu.VMEM((N_NODES,), jnp.int32))

    def body(*refs):
        if use_cids:
            (table, src, dst, cids, zeros, out, acc, cncpt_v,
             srcA, dstA, srcB, dstB, rowA, rowB, gsA, gsB) = refs
        else:
            (table, src, dst, zeros, out, acc,
             srcA, dstA, srcB, dstB, rowA, rowB, gsA, gsB) = refs

        c = lax.axis_index("c")
        s = lax.axis_index("s")
        t = c * NS + s
        e_base = t * (CH_PER_TILE * CHUNK)

        for k in range(ROW_CHUNKS_PER_SUB):
            row0 = (s * ROW_CHUNKS_PER_SUB + k) * ROW_CHUNK
            pltpu.sync_copy(zeros, acc.at[pl.ds(row0, ROW_CHUNK)])
        if use_cids:
            pltpu.sync_copy(cids, cncpt_v)

        plsc.subcore_barrier()

        def load_idx(x, sbuf, dbuf):
            off = e_base + x * CHUNK
            pltpu.sync_copy(src.at[pl.ds(off, CHUNK)], sbuf)
            pltpu.sync_copy(dst.at[pl.ds(off, CHUNK)], dbuf)
            if use_cids:
                for kk in range(CHUNK // L):
                    sl = pl.ds(kk * L, L)
                    sbuf[sl] = plsc.load_gather(cncpt_v, [sbuf[sl]])

        def step(i, carry):
            a = i * 2
            load_idx(a, srcA, dstA)
            ga = pltpu.async_copy(table.at[srcA], rowA, gsA)
            ga.wait()
            pltpu.sync_copy(rowA, acc.at[dstA], add=True)
            load_idx(a + 1, srcB, dstB)
            gb = pltpu.async_copy(table.at[srcB], rowB, gsB)
            gb.wait()
            pltpu.sync_copy(rowB, acc.at[dstB], add=True)
            return carry

        lax.fori_loop(0, CH_PER_TILE // 2, step, 0)
        plsc.subcore_barrier()

        for k in range(ROW_CHUNKS_PER_SUB):
            row0 = (s * ROW_CHUNKS_PER_SUB + k) * ROW_CHUNK
            pltpu.sync_copy(acc.at[pl.ds(row0, ROW_CHUNK)], out.at[c, pl.ds(row0, ROW_CHUNK)])

    return pl.kernel(
        body,
        out_type=jax.ShapeDtypeStruct((NC, N_PAD, D), jnp.float32),
        mesh=mesh,
        scratch_types=scratch,
        compiler_params=pltpu.CompilerParams(needs_layout_passes=False),
        name="edge_agg_cids" if use_cids else "edge_agg",
    )


def _linear_relu_body(p_ref, w_ref, b_ref, o_ref):
    x = p_ref[0] + p_ref[1]
    y = jnp.dot(x, w_ref[...], preferred_element_type=jnp.float32) + b_ref[...]
    o_ref[...] = jnp.maximum(y, 0.0)


def _linear_relu(parts, W, b):
    BN = 2000
    return pl.pallas_call(
        _linear_relu_body,
        grid=(N_NODES // BN,),
        in_specs=[
            pl.BlockSpec((NC, BN, D), lambda i: (0, i, 0)),
            pl.BlockSpec((D, D), lambda i: (0, 0)),
            pl.BlockSpec((1, D), lambda i: (0, 0)),
        ],
        out_specs=pl.BlockSpec((BN, D), lambda i: (i, 0)),
        out_shape=jax.ShapeDtypeStruct((N_NODES, D), jnp.float32),
    )(parts, W, b.reshape(1, D))


@jax.jit
def kernel(cncpt_ids, edge_index, emb, W1, b1, W2, b2):
    npad = E_PAD - N_EDGES
    src = jnp.concatenate([edge_index[0], jnp.zeros((npad,), jnp.int32)])
    dst = jnp.concatenate([edge_index[1], jnp.full((npad,), N_NODES, jnp.int32)])
    zeros = jnp.zeros((ROW_CHUNK, D), jnp.float32)

    agg1 = _make_edge_agg(True)(emb, src, dst, cncpt_ids, zeros)
    h1 = _linear_relu(agg1, W1, b1)
    agg2 = _make_edge_agg(False)(h1, src, dst, zeros)
    h2 = _linear_relu(agg2, W2, b2)
    return h2


# speedup vs baseline: 2.3279x; 2.3279x over previous
import jax
import jax.numpy as jnp
from jax import lax
from jax.experimental import pallas as pl
from jax.experimental.pallas import tpu as pltpu
from jax.experimental.pallas import tpu_sc as plsc

N_NODES = 10000
N_EDGES = 320000
D = 128

NC = 2
NS = 16
L = 16

CHUNK = 128
EDGES_PER_CORE = N_EDGES // NC
CHUNKS_PER_CORE = EDGES_PER_CORE // CHUNK
CHUNKS_BASE = CHUNKS_PER_CORE // NS
CHUNKS_REM = CHUNKS_PER_CORE % NS

N_PAD = 10240
ROW_CHUNK = 128
ROW_CHUNKS_PER_SUB = N_PAD // NS // ROW_CHUNK


def _make_edge_agg(use_cids: bool):
    mesh = plsc.VectorSubcoreMesh(
        core_axis_name="c", subcore_axis_name="s", num_cores=NC, num_subcores=NS
    )

    scratch = [
        pltpu.VMEM_SHARED((N_PAD, D), jnp.float32),
        pltpu.VMEM((CHUNK,), jnp.int32),
        pltpu.VMEM((CHUNK,), jnp.int32),
        pltpu.VMEM((CHUNK,), jnp.int32),
        pltpu.VMEM((CHUNK, D), jnp.float32),
        pltpu.SemaphoreType.DMA,
    ]
    if use_cids:
        scratch.insert(1, pltpu.VMEM((N_NODES,), jnp.int32))

    def body(*refs):
        if use_cids:
            (table, src, dst, cids, zeros, out,
             acc, cncpt_v, src_v, dst_v, cid_v, rows_v, sem) = refs
        else:
            (table, src, dst, zeros, out,
             acc, src_v, dst_v, cid_v, rows_v, sem) = refs

        c = lax.axis_index("c")
        s = lax.axis_index("s")

        for k in range(ROW_CHUNKS_PER_SUB):
            row0 = (s * ROW_CHUNKS_PER_SUB + k) * ROW_CHUNK
            pltpu.sync_copy(zeros, acc.at[pl.ds(row0, ROW_CHUNK)])
        if use_cids:
            pltpu.sync_copy(cids, cncpt_v)
        plsc.subcore_barrier()

        nloc = CHUNKS_BASE + jnp.where(s < CHUNKS_REM, 1, 0)

        def step(i, carry):
            chunk = i * NS + s
            base = c * EDGES_PER_CORE + chunk * CHUNK
            pltpu.sync_copy(src.at[pl.ds(base, CHUNK)], src_v)
            pltpu.sync_copy(dst.at[pl.ds(base, CHUNK)], dst_v)
            if use_cids:
                for j in range(CHUNK // L):
                    v = src_v[pl.ds(j * L, L)]
                    cid_v[pl.ds(j * L, L)] = plsc.load_gather(cncpt_v, [v])
                idx = cid_v
            else:
                idx = src_v
            pltpu.async_copy(table.at[idx], rows_v, sem).wait()
            pltpu.sync_copy(rows_v, acc.at[dst_v], add=True)
            return carry

        lax.fori_loop(0, nloc, step, 0)
        plsc.subcore_barrier()

        for k in range(ROW_CHUNKS_PER_SUB):
            row0 = (s * ROW_CHUNKS_PER_SUB + k) * ROW_CHUNK
            pltpu.sync_copy(acc.at[pl.ds(row0, ROW_CHUNK)], out.at[c, pl.ds(row0, ROW_CHUNK)])

    return pl.kernel(
        body,
        out_type=jax.ShapeDtypeStruct((NC, N_PAD, D), jnp.float32),
        mesh=mesh,
        scratch_types=scratch,
        compiler_params=pltpu.CompilerParams(needs_layout_passes=False),
        name="edge_agg_cids" if use_cids else "edge_agg",
    )


def _linear_relu_body(p_ref, w_ref, b_ref, o_ref):
    x = p_ref[0] + p_ref[1]
    y = jnp.dot(x, w_ref[...], preferred_element_type=jnp.float32) + b_ref[...]
    o_ref[...] = jnp.maximum(y, 0.0)


def _linear_relu(parts, W, b):
    BN = 2000
    return pl.pallas_call(
        _linear_relu_body,
        grid=(N_NODES // BN,),
        in_specs=[
            pl.BlockSpec((NC, BN, D), lambda i: (0, i, 0)),
            pl.BlockSpec((D, D), lambda i: (0, 0)),
            pl.BlockSpec((1, D), lambda i: (0, 0)),
        ],
        out_specs=pl.BlockSpec((BN, D), lambda i: (i, 0)),
        out_shape=jax.ShapeDtypeStruct((N_NODES, D), jnp.float32),
    )(parts, W, b.reshape(1, D))


@jax.jit
def kernel(cncpt_ids, edge_index, emb, W1, b1, W2, b2):
    src = edge_index[0]
    dst = edge_index[1]
    zeros = jnp.zeros((ROW_CHUNK, D), jnp.float32)

    agg1 = _make_edge_agg(True)(emb, src, dst, cncpt_ids, zeros)
    h1 = _linear_relu(agg1[:, :N_NODES], W1, b1)
    agg2 = _make_edge_agg(False)(h1, src, dst, zeros)
    h2 = _linear_relu(agg2[:, :N_NODES], W2, b2)
    return h2


# speedup vs baseline: 2.7040x; 1.1615x over previous
import jax
import jax.numpy as jnp
from jax import lax
from jax.experimental import pallas as pl
from jax.experimental.pallas import tpu as pltpu
from jax.experimental.pallas import tpu_sc as plsc

N_NODES = 10000
N_EDGES = 320000
D = 128

NC = 2
NS = 16
L = 16

CHUNK = 128
EDGES_PER_CORE = N_EDGES // NC
CHUNKS_PER_CORE = EDGES_PER_CORE // CHUNK
CHUNKS_BASE = CHUNKS_PER_CORE // NS
CHUNKS_REM = CHUNKS_PER_CORE % NS

N_PAD = 10240
ROW_CHUNK = 128
ROW_CHUNKS_PER_SUB = N_PAD // NS // ROW_CHUNK


def _make_edge_agg(use_cids: bool):
    mesh = plsc.VectorSubcoreMesh(
        core_axis_name="c", subcore_axis_name="s", num_cores=NC, num_subcores=NS
    )

    scratch = [
        pltpu.VMEM_SHARED((N_PAD, D), jnp.float32),
        pltpu.VMEM((CHUNK,), jnp.int32),
        pltpu.VMEM((CHUNK,), jnp.int32),
        pltpu.VMEM((CHUNK,), jnp.int32),
        pltpu.VMEM((CHUNK, D), jnp.float32),
        pltpu.SemaphoreType.DMA,
    ]
    if use_cids:
        scratch.insert(1, pltpu.VMEM((N_NODES,), jnp.int32))

    def body(*refs):
        if use_cids:
            (table, src, dst, cids, zeros, out,
             acc, cncpt_v, src_v, dst_v, cid_v, rows_v, sem) = refs
        else:
            (table, src, dst, zeros, out,
             acc, src_v, dst_v, cid_v, rows_v, sem) = refs

        c = lax.axis_index("c")
        s = lax.axis_index("s")

        for k in range(ROW_CHUNKS_PER_SUB):
            row0 = (s * ROW_CHUNKS_PER_SUB + k) * ROW_CHUNK
            pltpu.sync_copy(zeros, acc.at[pl.ds(row0, ROW_CHUNK)])
        if use_cids:
            pltpu.sync_copy(cids, cncpt_v)
        plsc.subcore_barrier()

        nloc = CHUNKS_BASE + jnp.where(s < CHUNKS_REM, 1, 0)

        def step(i, carry):
            chunk = i * NS + s
            base = c * EDGES_PER_CORE + chunk * CHUNK
            pltpu.sync_copy(src.at[pl.ds(base, CHUNK)], src_v)
            pltpu.sync_copy(dst.at[pl.ds(base, CHUNK)], dst_v)
            if use_cids:
                for j in range(CHUNK // L):
                    v = src_v[pl.ds(j * L, L)]
                    cid_v[pl.ds(j * L, L)] = plsc.load_gather(cncpt_v, [v])
                idx = cid_v
            else:
                idx = src_v
            pltpu.async_copy(table.at[idx], rows_v, sem).wait()
            pltpu.sync_copy(rows_v, acc.at[dst_v], add=True)
            return carry

        lax.fori_loop(0, nloc, step, 0)
        plsc.subcore_barrier()

        for k in range(ROW_CHUNKS_PER_SUB):
            row0 = (s * ROW_CHUNKS_PER_SUB + k) * ROW_CHUNK
            pltpu.sync_copy(acc.at[pl.ds(row0, ROW_CHUNK)], out.at[c, pl.ds(row0, ROW_CHUNK)])

    return pl.kernel(
        body,
        out_type=jax.ShapeDtypeStruct((NC, N_PAD, D), jnp.float32),
        mesh=mesh,
        scratch_types=scratch,
        compiler_params=pltpu.CompilerParams(needs_layout_passes=False),
        name="edge_agg_cids" if use_cids else "edge_agg",
    )


def _linear_relu_body(p_ref, w_ref, b_ref, o_ref):
    x = p_ref[0] + p_ref[1]
    y = jnp.dot(x, w_ref[...], preferred_element_type=jnp.float32) + b_ref[...]
    o_ref[...] = jnp.maximum(y, 0.0)


def _linear_relu(parts, W, b):
    BN = 2000
    return pl.pallas_call(
        _linear_relu_body,
        grid=(N_NODES // BN,),
        in_specs=[
            pl.BlockSpec((NC, BN, D), lambda i: (0, i, 0)),
            pl.BlockSpec((D, D), lambda i: (0, 0)),
            pl.BlockSpec((1, D), lambda i: (0, 0)),
        ],
        out_specs=pl.BlockSpec((BN, D), lambda i: (i, 0)),
        out_shape=jax.ShapeDtypeStruct((N_NODES, D), jnp.float32),
    )(parts, W, b.reshape(1, D))


@jax.jit
def kernel(cncpt_ids, edge_index, emb, W1, b1, W2, b2):
    src = edge_index[0]
    dst = edge_index[1]
    zeros = jnp.zeros((ROW_CHUNK, D), jnp.float32)

    agg1 = _make_edge_agg(True)(emb, src, dst, cncpt_ids, zeros)
    h1 = _linear_relu(agg1, W1, b1)
    agg2 = _make_edge_agg(False)(h1, src, dst, zeros)
    h2 = _linear_relu(agg2, W2, b2)
    return h2
